# baseline (speedup 1.0000x reference)
import functools

import jax
import jax.numpy as jnp
from jax import lax
from jax.experimental import pallas as pl
from jax.experimental.pallas import tpu as pltpu
from jax.experimental.pallas import tpu_sc as plsc

N = 10000
D_IN = 128
H1 = 64
H2 = 32
D_OUT = 10
G = 64

NW = 32
CHUNK = 128
CPT = 80
CPTA = 160
EP = NW * CPT * CHUNK
ND = 10240
RPT = ND // 16


def _mesh():
    return plsc.VectorSubcoreMesh(core_axis_name="c", subcore_axis_name="s",
                                  num_cores=2, num_subcores=16)


_SC_PARAMS = pltpu.CompilerParams(use_tc_tiling_on_sc=False)


def _deg_sc(dst2d, ones_hbm, zeros_hbm):
    @functools.partial(
        pl.kernel,
        out_type=jax.ShapeDtypeStruct((2, ND, 1), jnp.float32),
        mesh=_mesh(),
        compiler_params=_SC_PARAMS,
        scratch_types=[
            pltpu.VMEM((CPT, CHUNK), jnp.int32),
            pltpu.VMEM((CHUNK, 1), jnp.float32),
            pltpu.VMEM_SHARED((ND, 1), jnp.float32),
        ],
    )
    def k(dst_hbm, ones_h, zeros_h, deg_hbm, dstv, ones_v, degs):
        c = lax.axis_index("c")
        s = lax.axis_index("s")
        wid = c * 16 + s
        row0 = s * RPT
        pltpu.sync_copy(zeros_h.at[pl.ds(row0, RPT)], degs.at[pl.ds(row0, RPT)])
        pltpu.sync_copy(ones_h, ones_v)
        pltpu.sync_copy(dst_hbm.at[pl.ds(wid * CPT, CPT)], dstv)
        plsc.subcore_barrier()

        @pl.loop(0, CPT)
        def _(j):
            pltpu.sync_copy(ones_v, degs.at[dstv.at[j]], add=True)

        plsc.subcore_barrier()
        pltpu.sync_copy(degs.at[pl.ds(row0, RPT)],
                        deg_hbm.at[c, pl.ds(row0, RPT)])

    return k(dst2d, ones_hbm, zeros_hbm)


def _agg_sc(phalf, src2d, dst2d, H):
    HH = H // 2
    NPT = N // 16

    @functools.partial(
        pl.kernel,
        out_type=jax.ShapeDtypeStruct((2, ND, HH), jnp.float32),
        mesh=_mesh(),
        compiler_params=_SC_PARAMS,
        scratch_types=[
            pltpu.VMEM((CPTA, CHUNK), jnp.int32),
            pltpu.VMEM((CPTA, CHUNK), jnp.int32),
            pltpu.VMEM((4, CHUNK, HH), jnp.float32),
            pltpu.VMEM((64, HH), jnp.float32),
            pltpu.VMEM_SHARED((N, HH), jnp.float32),
            pltpu.VMEM_SHARED((ND, HH), jnp.float32),
            pltpu.SemaphoreType.DMA,
            pltpu.SemaphoreType.DMA,
        ],
    )
    def k(p_hbm, src_hbm, dst_hbm, q_hbm, srcv, dstv, rows, zbuf, ps, qs,
          sem0, sem1):
        c = lax.axis_index("c")
        s = lax.axis_index("s")
        row0 = s * RPT
        base = s * CPTA

        @pl.loop(0, 64)
        def _(i):
            @pl.loop(0, HH, step=16)
            def _(kk):
                zbuf[i, pl.ds(kk, 16)] = jnp.zeros((16,), jnp.float32)

        @pl.loop(0, RPT, step=64)
        def _(r):
            pltpu.sync_copy(zbuf, qs.at[pl.ds(row0 + r, 64)])

        pltpu.sync_copy(p_hbm.at[c, pl.ds(s * NPT, NPT)],
                        ps.at[pl.ds(s * NPT, NPT)])
        pltpu.sync_copy(src_hbm.at[pl.ds(base, CPTA)], srcv)
        pltpu.sync_copy(dst_hbm.at[pl.ds(base, CPTA)], dstv)
        plsc.subcore_barrier()

        for b in range(4):
            pltpu.async_copy(ps.at[srcv.at[b]], rows.at[b], sem0)

        def _refill(j, b):
            @pl.when(j + 4 + b < CPTA)
            def _():
                pltpu.async_copy(ps.at[srcv.at[j + 4 + b]], rows.at[b],
                                 sem0)

        @pl.loop(0, CPTA, step=4)
        def _(j):
            scats = []
            for b in range(4):
                pltpu.make_async_copy(ps.at[srcv.at[j + b]], rows.at[b],
                                      sem0).wait()
                scats.append(pltpu.async_copy(rows.at[b],
                                              qs.at[dstv.at[j + b]],
                                              sem1, add=True))
            for b in range(4):
                scats[b].wait()
                _refill(j, b)

        plsc.subcore_barrier()
        pltpu.sync_copy(qs.at[pl.ds(row0, RPT)],
                        q_hbm.at[c, pl.ds(row0, RPT)])

    return k(phalf, src2d, dst2d)


_BLK = 2000
_NBLK = N // _BLK


def _tc_mm1(x, W1):
    def body(x_ref, w_ref, h_ref):
        h_ref[...] = jnp.dot(x_ref[...], w_ref[...],
                             preferred_element_type=jnp.float32)

    return pl.pallas_call(
        body,
        grid=(_NBLK,),
        in_specs=[
            pl.BlockSpec((_BLK, D_IN), lambda i: (i, 0)),
            pl.BlockSpec((D_IN, H1), lambda i: (0, 0)),
        ],
        out_specs=pl.BlockSpec((_BLK, H1), lambda i: (i, 0)),
        out_shape=jax.ShapeDtypeStruct((N, H1), jnp.float32),
    )(x, W1)


def _tc_scale1(h, deg_part):
    def body(h_ref, d_ref, p_ref, dinv_ref):
        deg = d_ref[0] + d_ref[1] + 1.0
        dinv = lax.rsqrt(deg)
        p = h_ref[...] * dinv
        p_ref[0] = p[:, :H1 // 2]
        p_ref[1] = p[:, H1 // 2:]
        dinv_ref[...] = dinv

    return pl.pallas_call(
        body,
        grid=(_NBLK,),
        in_specs=[
            pl.BlockSpec((_BLK, H1), lambda i: (i, 0)),
            pl.BlockSpec((2, _BLK, 1), lambda i: (0, i, 0)),
        ],
        out_specs=[
            pl.BlockSpec((2, _BLK, H1 // 2), lambda i: (0, i, 0)),
            pl.BlockSpec((_BLK, 1), lambda i: (i, 0)),
        ],
        out_shape=[
            jax.ShapeDtypeStruct((2, N, H1 // 2), jnp.float32),
            jax.ShapeDtypeStruct((N, 1), jnp.float32),
        ],
    )(h, deg_part)


def _tc2(q1, p1, dinv, b1, W2):
    def body(q_ref, p_ref, d_ref, b_ref, w_ref, o_ref):
        dinv = d_ref[...]
        qp = jnp.concatenate([q_ref[0] + p_ref[0], q_ref[1] + p_ref[1]],
                             axis=1)
        z = qp * dinv + b_ref[...]
        h1 = jnp.maximum(z, 0.0)
        h2 = jnp.dot(h1, w_ref[...],
                     preferred_element_type=jnp.float32)
        p2 = h2 * dinv
        o_ref[0] = p2[:, :H2 // 2]
        o_ref[1] = p2[:, H2 // 2:]

    return pl.pallas_call(
        body,
        grid=(_NBLK,),
        in_specs=[
            pl.BlockSpec((2, _BLK, H1 // 2), lambda i: (0, i, 0)),
            pl.BlockSpec((2, _BLK, H1 // 2), lambda i: (0, i, 0)),
            pl.BlockSpec((_BLK, 1), lambda i: (i, 0)),
            pl.BlockSpec((1, H1), lambda i: (0, 0)),
            pl.BlockSpec((H1, H2), lambda i: (0, 0)),
        ],
        out_specs=pl.BlockSpec((2, _BLK, H2 // 2), lambda i: (0, i, 0)),
        out_shape=jax.ShapeDtypeStruct((2, N, H2 // 2), jnp.float32),
    )(q1, p1, dinv, b1, W2)


def _tc3(q2, p2, dinv, b2, batch2d, Wfc, bfc):
    def body(q_ref, p_ref, d_ref, b_ref, bat_ref, w_ref, bf_ref, o_ref, pooled):
        i = pl.program_id(0)

        @pl.when(i == 0)
        def _():
            pooled[...] = jnp.full((G // 4, 4 * H2), -jnp.inf, jnp.float32)

        qp = jnp.concatenate([q_ref[0] + p_ref[0], q_ref[1] + p_ref[1]],
                             axis=1)
        z = qp * d_ref[...] + b_ref[...]
        h2 = jnp.maximum(z, 0.0)
        bat = bat_ref[...]
        ht = jnp.concatenate([h2, h2, h2, h2], axis=1)
        lane_g = lax.broadcasted_iota(jnp.int32, (1, 4 * H2), 1) // H2
        parts = []
        for r in range(G // 4):
            m = bat == (4 * r + lane_g)
            cand = jnp.where(m, ht, -jnp.inf)
            parts.append(jnp.max(cand, axis=0, keepdims=True))
        blk_pool = jnp.concatenate(parts, axis=0)
        pooled[...] = jnp.maximum(pooled[...], blk_pool)

        @pl.when(i == _NBLK - 1)
        def _():
            o_ref[...] = jnp.dot(pooled[...], w_ref[...],
                                 preferred_element_type=jnp.float32) + bf_ref[...]

    return pl.pallas_call(
        body,
        grid=(_NBLK,),
        in_specs=[
            pl.BlockSpec((2, _BLK, H2 // 2), lambda i: (0, i, 0)),
            pl.BlockSpec((2, _BLK, H2 // 2), lambda i: (0, i, 0)),
            pl.BlockSpec((_BLK, 1), lambda i: (i, 0)),
            pl.BlockSpec((1, H2), lambda i: (0, 0)),
            pl.BlockSpec((_BLK, 1), lambda i: (i, 0)),
            pl.BlockSpec((4 * H2, 4 * D_OUT), lambda i: (0, 0)),
            pl.BlockSpec((1, 4 * D_OUT), lambda i: (0, 0)),
        ],
        out_specs=pl.BlockSpec((G // 4, 4 * D_OUT), lambda i: (0, 0)),
        out_shape=jax.ShapeDtypeStruct((G // 4, 4 * D_OUT), jnp.float32),
        scratch_shapes=[pltpu.VMEM((G // 4, 4 * H2), jnp.float32)],
    )(q2, p2, dinv, b2, batch2d, Wfc, bfc)


def kernel(x, edge_index, batch, W1, b1, W2, b2, Wfc, bfc):
    x = x.astype(jnp.float32)
    src = edge_index[0].astype(jnp.int32)
    dst = edge_index[1].astype(jnp.int32)
    e = src.shape[0]
    pad = EP - e
    src2d = jnp.concatenate(
        [src, jnp.zeros((pad,), jnp.int32)]).reshape(EP // CHUNK, CHUNK)
    dst2d = jnp.concatenate(
        [dst, jnp.full((pad,), N, jnp.int32)]).reshape(EP // CHUNK, CHUNK)
    batch2d = batch.astype(jnp.int32).reshape(N, 1)

    ones16 = jnp.ones((CHUNK, 1), jnp.float32)
    zeros16 = jnp.zeros((ND, 1), jnp.float32)

    h1pre = _tc_mm1(x, W1)
    deg_part = _deg_sc(dst2d, ones16, zeros16)
    p1, dinv = _tc_scale1(h1pre, deg_part)
    q1 = _agg_sc(p1, src2d, dst2d, H1)
    p2 = _tc2(q1, p1, dinv, b1.reshape(1, H1), W2)
    q2 = _agg_sc(p2, src2d, dst2d, H2)
    wblk = jnp.zeros((4 * H2, 4 * D_OUT), jnp.float32)
    for kk in range(4):
        wblk = wblk.at[kk * H2:(kk + 1) * H2,
                       kk * D_OUT:(kk + 1) * D_OUT].set(Wfc)
    bfb = jnp.tile(bfc.reshape(1, D_OUT), (1, 4))
    out4 = _tc3(q2, p2, dinv, b2.reshape(1, H2), batch2d, wblk, bfb)
    return out4.reshape(G, D_OUT)


# speedup vs baseline: 1.0033x; 1.0033x over previous
import functools

import jax
import jax.numpy as jnp
from jax import lax
from jax.experimental import pallas as pl
from jax.experimental.pallas import tpu as pltpu
from jax.experimental.pallas import tpu_sc as plsc

N = 10000
D_IN = 128
H1 = 64
H2 = 32
D_OUT = 10
G = 64

NW = 32
CHUNK = 128
CPT = 80
CPTA = 160
EP = NW * CPT * CHUNK
ND = 10240
RPT = ND // 16


def _mesh():
    return plsc.VectorSubcoreMesh(core_axis_name="c", subcore_axis_name="s",
                                  num_cores=2, num_subcores=16)


_SC_PARAMS = pltpu.CompilerParams(use_tc_tiling_on_sc=False)


def _deg_sc(dst2d, ones_hbm, zeros_hbm):
    @functools.partial(
        pl.kernel,
        out_type=jax.ShapeDtypeStruct((2, ND, 1), jnp.float32),
        mesh=_mesh(),
        compiler_params=_SC_PARAMS,
        scratch_types=[
            pltpu.VMEM((CPT, CHUNK), jnp.int32),
            pltpu.VMEM((CHUNK, 1), jnp.float32),
            pltpu.VMEM_SHARED((ND, 1), jnp.float32),
        ],
    )
    def k(dst_hbm, ones_h, zeros_h, deg_hbm, dstv, ones_v, degs):
        c = lax.axis_index("c")
        s = lax.axis_index("s")
        wid = c * 16 + s
        row0 = s * RPT
        pltpu.sync_copy(zeros_h.at[pl.ds(row0, RPT)], degs.at[pl.ds(row0, RPT)])
        pltpu.sync_copy(ones_h, ones_v)
        pltpu.sync_copy(dst_hbm.at[pl.ds(wid * CPT, CPT)], dstv)
        plsc.subcore_barrier()

        @pl.loop(0, CPT)
        def _(j):
            pltpu.sync_copy(ones_v, degs.at[dstv.at[j]], add=True)

        plsc.subcore_barrier()
        pltpu.sync_copy(degs.at[pl.ds(row0, RPT)],
                        deg_hbm.at[c, pl.ds(row0, RPT)])

    return k(dst2d, ones_hbm, zeros_hbm)


def _agg_sc(phalf, src2d, dst2d, H):
    HH = H // 2
    NPT = N // 16

    @functools.partial(
        pl.kernel,
        out_type=jax.ShapeDtypeStruct((2, ND, HH), jnp.float32),
        mesh=_mesh(),
        compiler_params=_SC_PARAMS,
        scratch_types=[
            pltpu.VMEM((CPTA, CHUNK), jnp.int32),
            pltpu.VMEM((CPTA, CHUNK), jnp.int32),
            pltpu.VMEM((4, CHUNK, HH), jnp.float32),
            pltpu.VMEM((64, HH), jnp.float32),
            pltpu.VMEM_SHARED((N, HH), jnp.float32),
            pltpu.VMEM_SHARED((ND, HH), jnp.float32),
            pltpu.SemaphoreType.DMA,
            pltpu.SemaphoreType.DMA,
        ],
    )
    def k(p_hbm, src_hbm, dst_hbm, q_hbm, srcv, dstv, rows, zbuf, ps, qs,
          sem0, sem1):
        c = lax.axis_index("c")
        s = lax.axis_index("s")
        row0 = s * RPT
        base = s * CPTA

        @pl.loop(0, 64)
        def _(i):
            @pl.loop(0, HH, step=16)
            def _(kk):
                zbuf[i, pl.ds(kk, 16)] = jnp.zeros((16,), jnp.float32)

        @pl.loop(0, RPT, step=64)
        def _(r):
            pltpu.sync_copy(zbuf, qs.at[pl.ds(row0 + r, 64)])

        pltpu.sync_copy(p_hbm.at[c, pl.ds(s * NPT, NPT)],
                        ps.at[pl.ds(s * NPT, NPT)])
        pltpu.sync_copy(src_hbm.at[pl.ds(base, CPTA)], srcv)
        pltpu.sync_copy(dst_hbm.at[pl.ds(base, CPTA)], dstv)
        plsc.subcore_barrier()

        for b in range(4):
            pltpu.async_copy(ps.at[srcv.at[b]], rows.at[b], sem0)

        def _refill(j, b):
            @pl.when(j + 4 + b < CPTA)
            def _():
                pltpu.async_copy(ps.at[srcv.at[j + 4 + b]], rows.at[b],
                                 sem0)

        @pl.loop(0, CPTA, step=4)
        def _(j):
            scats = []
            for b in range(4):
                pltpu.make_async_copy(ps.at[srcv.at[j + b]], rows.at[b],
                                      sem0).wait()
                scats.append(pltpu.async_copy(rows.at[b],
                                              qs.at[dstv.at[j + b]],
                                              sem1, add=True))
            for b in range(4):
                scats[b].wait()
                _refill(j, b)

        plsc.subcore_barrier()
        pltpu.sync_copy(qs.at[pl.ds(row0, RPT)],
                        q_hbm.at[c, pl.ds(row0, RPT)])

    return k(phalf, src2d, dst2d)


_BLK = 2000
_NBLK = N // _BLK


def _tc1(x, W1, deg_part):
    def body(x_ref, w_ref, d_ref, p_ref, dinv_ref):
        deg = d_ref[0] + d_ref[1] + 1.0
        dinv = lax.rsqrt(deg)
        h = jnp.dot(x_ref[...], w_ref[...],
                    preferred_element_type=jnp.float32)
        p = h * dinv
        p_ref[0] = p[:, :H1 // 2]
        p_ref[1] = p[:, H1 // 2:]
        dinv_ref[...] = dinv

    return pl.pallas_call(
        body,
        grid=(_NBLK,),
        in_specs=[
            pl.BlockSpec((_BLK, D_IN), lambda i: (i, 0)),
            pl.BlockSpec((D_IN, H1), lambda i: (0, 0)),
            pl.BlockSpec((2, _BLK, 1), lambda i: (0, i, 0)),
        ],
        out_specs=[
            pl.BlockSpec((2, _BLK, H1 // 2), lambda i: (0, i, 0)),
            pl.BlockSpec((_BLK, 1), lambda i: (i, 0)),
        ],
        out_shape=[
            jax.ShapeDtypeStruct((2, N, H1 // 2), jnp.float32),
            jax.ShapeDtypeStruct((N, 1), jnp.float32),
        ],
    )(x, W1, deg_part)


def _tc2(q1, p1, dinv, b1, W2):
    def body(q_ref, p_ref, d_ref, b_ref, w_ref, o_ref):
        dinv = d_ref[...]
        qp = jnp.concatenate([q_ref[0] + p_ref[0], q_ref[1] + p_ref[1]],
                             axis=1)
        z = qp * dinv + b_ref[...]
        h1 = jnp.maximum(z, 0.0)
        h2 = jnp.dot(h1, w_ref[...],
                     preferred_element_type=jnp.float32)
        p2 = h2 * dinv
        o_ref[0] = p2[:, :H2 // 2]
        o_ref[1] = p2[:, H2 // 2:]

    return pl.pallas_call(
        body,
        grid=(_NBLK,),
        in_specs=[
            pl.BlockSpec((2, _BLK, H1 // 2), lambda i: (0, i, 0)),
            pl.BlockSpec((2, _BLK, H1 // 2), lambda i: (0, i, 0)),
            pl.BlockSpec((_BLK, 1), lambda i: (i, 0)),
            pl.BlockSpec((1, H1), lambda i: (0, 0)),
            pl.BlockSpec((H1, H2), lambda i: (0, 0)),
        ],
        out_specs=pl.BlockSpec((2, _BLK, H2 // 2), lambda i: (0, i, 0)),
        out_shape=jax.ShapeDtypeStruct((2, N, H2 // 2), jnp.float32),
    )(q1, p1, dinv, b1, W2)


def _tc3(q2, p2, dinv, b2, batch2d, Wfc, bfc):
    def body(q_ref, p_ref, d_ref, b_ref, bat_ref, w_ref, bf_ref, o_ref, pooled):
        i = pl.program_id(0)

        @pl.when(i == 0)
        def _():
            pooled[...] = jnp.full((G // 4, 4 * H2), -jnp.inf, jnp.float32)

        qp = jnp.concatenate([q_ref[0] + p_ref[0], q_ref[1] + p_ref[1]],
                             axis=1)
        z = qp * d_ref[...] + b_ref[...]
        h2 = jnp.maximum(z, 0.0)
        bat = bat_ref[...]
        ht = jnp.concatenate([h2, h2, h2, h2], axis=1)
        lane_g = lax.broadcasted_iota(jnp.int32, (1, 4 * H2), 1) // H2
        parts = []
        for r in range(G // 4):
            m = bat == (4 * r + lane_g)
            cand = jnp.where(m, ht, -jnp.inf)
            parts.append(jnp.max(cand, axis=0, keepdims=True))
        blk_pool = jnp.concatenate(parts, axis=0)
        pooled[...] = jnp.maximum(pooled[...], blk_pool)

        @pl.when(i == _NBLK - 1)
        def _():
            o_ref[...] = jnp.dot(pooled[...], w_ref[...],
                                 preferred_element_type=jnp.float32) + bf_ref[...]

    return pl.pallas_call(
        body,
        grid=(_NBLK,),
        in_specs=[
            pl.BlockSpec((2, _BLK, H2 // 2), lambda i: (0, i, 0)),
            pl.BlockSpec((2, _BLK, H2 // 2), lambda i: (0, i, 0)),
            pl.BlockSpec((_BLK, 1), lambda i: (i, 0)),
            pl.BlockSpec((1, H2), lambda i: (0, 0)),
            pl.BlockSpec((_BLK, 1), lambda i: (i, 0)),
            pl.BlockSpec((4 * H2, 4 * D_OUT), lambda i: (0, 0)),
            pl.BlockSpec((1, 4 * D_OUT), lambda i: (0, 0)),
        ],
        out_specs=pl.BlockSpec((G // 4, 4 * D_OUT), lambda i: (0, 0)),
        out_shape=jax.ShapeDtypeStruct((G // 4, 4 * D_OUT), jnp.float32),
        scratch_shapes=[pltpu.VMEM((G // 4, 4 * H2), jnp.float32)],
    )(q2, p2, dinv, b2, batch2d, Wfc, bfc)


def kernel(x, edge_index, batch, W1, b1, W2, b2, Wfc, bfc):
    x = x.astype(jnp.float32)
    src = edge_index[0].astype(jnp.int32)
    dst = edge_index[1].astype(jnp.int32)
    e = src.shape[0]
    pad = EP - e
    src2d = jnp.concatenate(
        [src, jnp.zeros((pad,), jnp.int32)]).reshape(EP // CHUNK, CHUNK)
    dst2d = jnp.concatenate(
        [dst, jnp.full((pad,), N, jnp.int32)]).reshape(EP // CHUNK, CHUNK)
    batch2d = batch.astype(jnp.int32).reshape(N, 1)

    ones16 = jnp.ones((CHUNK, 1), jnp.float32)
    zeros16 = jnp.zeros((ND, 1), jnp.float32)

    deg_part = _deg_sc(dst2d, ones16, zeros16)
    p1, dinv = _tc1(x, W1, deg_part)
    q1 = _agg_sc(p1, src2d, dst2d, H1)
    p2 = _tc2(q1, p1, dinv, b1.reshape(1, H1), W2)
    q2 = _agg_sc(p2, src2d, dst2d, H2)
    wblk = jnp.zeros((4 * H2, 4 * D_OUT), jnp.float32)
    for kk in range(4):
        wblk = wblk.at[kk * H2:(kk + 1) * H2,
                       kk * D_OUT:(kk + 1) * D_OUT].set(Wfc)
    bfb = jnp.tile(bfc.reshape(1, D_OUT), (1, 4))
    out4 = _tc3(q2, p2, dinv, b2.reshape(1, H2), batch2d, wblk, bfb)
    return out4.reshape(G, D_OUT)
